# baseline (speedup 1.0000x reference)
import jax
import jax.numpy as jnp
from jax import lax
from jax.experimental import pallas as pl
from jax.experimental.pallas import tpu as pltpu
from jax.experimental.pallas import tpu_sc as plsc

V = 1000000
E = 64
BINS = 30
B = 4096
Q = 20
QP = 24
D = 200
DPP = 208
DHP = DPP // 2
DP = 256

NQ_ROWS = B * QP
ND_ROWS = B * DPP
NW = 32
QROWS_PER_W = NQ_ROWS // NW
DROWS_PER_W = ND_ROWS // NW
GCHUNK = 1024
NQ_CHUNKS = QROWS_PER_W // GCHUNK
ND_CHUNKS = DROWS_PER_W // GCHUNK

BB = 8
NEG_BIG = -1e30


def _sc_gather_body(qidx_hbm, didx_hbm, table_hbm, outq_hbm, outd_hbm,
                    idx_v, rows_v, sem):
    wid = lax.axis_index("s") * 2 + lax.axis_index("c")

    def qchunk(i, carry):
        off = wid * QROWS_PER_W + i * GCHUNK
        pltpu.sync_copy(qidx_hbm.at[pl.ds(off, GCHUNK)], idx_v)
        pltpu.async_copy(table_hbm.at[idx_v], rows_v, sem).wait()
        pltpu.sync_copy(rows_v, outq_hbm.at[pl.ds(off, GCHUNK)])
        return carry

    def dchunk(i, carry):
        off = wid * DROWS_PER_W + i * GCHUNK
        half = GCHUNK // 2
        pltpu.sync_copy(didx_hbm.at[pl.ds(off, GCHUNK)], idx_v)
        pltpu.async_copy(table_hbm.at[idx_v], rows_v, sem).wait()
        pltpu.sync_copy(rows_v.at[pl.ds(0, half), :],
                        outd_hbm.at[pl.ds(off // 2, half), pl.ds(0, E)])
        pltpu.sync_copy(rows_v.at[pl.ds(half, half), :],
                        outd_hbm.at[pl.ds(off // 2, half), pl.ds(E, E)])
        return carry

    lax.fori_loop(0, NQ_CHUNKS, qchunk, 0, unroll=False)
    lax.fori_loop(0, ND_CHUNKS, dchunk, 0, unroll=False)


@jax.jit
def _sc_gather(qidx, didx, table):
    mesh = plsc.VectorSubcoreMesh(core_axis_name="c", subcore_axis_name="s")
    f = pl.kernel(
        _sc_gather_body,
        out_type=(
            jax.ShapeDtypeStruct((NQ_ROWS, E), jnp.float32),
            jax.ShapeDtypeStruct((ND_ROWS // 2, 2 * E), jnp.float32),
        ),
        mesh=mesh,
        compiler_params=pltpu.CompilerParams(use_tc_tiling_on_sc=False),
        scratch_types=[
            pltpu.VMEM((GCHUNK,), jnp.int32),
            pltpu.VMEM((GCHUNK, E), jnp.float32),
            pltpu.SemaphoreType.DMA,
        ],
    )
    return f(qidx, didx, table)


def _tc_body(qe_ref, de_ref, qt_ref, dtp_ref, sel2_ref, rhs2_ref, w_ref,
             mW1_ref, mb1_ref, mW2_ref, mb2_ref, gW1_ref, gb1_ref, gW2_ref,
             gb2_ref, out_ref, s_ref):
    qm = (qt_ref[...] > 1).astype(jnp.float32)
    qe = qe_ref[...]
    dep = de_ref[...]

    ones_e = jnp.ones((E, 8), jnp.float32)
    ones_1e = jnp.ones((1, E), jnp.float32)
    qnorm2 = lax.dot_general(qe * qe, ones_e, (((1,), (0,)), ((), ())),
                             preferred_element_type=jnp.float32)[:, 0:1]
    rq = qm * (1.0 / jnp.maximum(jnp.sqrt(qnorm2), 1e-13))
    qn = qe * lax.dot_general(rq, ones_1e, (((1,), (0,)), ((), ())),
                              preferred_element_type=jnp.float32)

    de2 = dep * dep
    dn2c = lax.dot_general(de2, rhs2_ref[...], (((1,), (0,)), ((), ())),
                           preferred_element_type=jnp.float32)[:, 0:2]
    dmc = (dtp_ref[...] > 1).astype(jnp.float32)
    rdc = dmc * (1.0 / jnp.maximum(jnp.sqrt(dn2c), 1e-13))
    scale = lax.dot_general(rdc, sel2_ref[...], (((1,), (0,)), ((), ())),
                            preferred_element_type=jnp.float32)
    dnp = dep * scale

    zq = jnp.zeros((BB * QP, E), jnp.float32)
    qn_l = jnp.concatenate([qn, zq], axis=1)
    qn_r = jnp.concatenate([zq, qn], axis=1)
    for i in range(BB):
        dni = dnp[i * DHP:(i + 1) * DHP, :]
        raw_l = lax.dot_general(qn_l[i * QP:(i + 1) * QP, :], dni,
                                (((1,), (1,)), ((), ())),
                                preferred_element_type=jnp.float32,
                                precision=lax.Precision.HIGHEST)
        raw_r = lax.dot_general(qn_r[i * QP:(i + 1) * QP, :], dni,
                                (((1,), (1,)), ((), ())),
                                preferred_element_type=jnp.float32,
                                precision=lax.Precision.HIGHEST)
        s_ref[i * QP:(i + 1) * QP, 0:DHP] = (raw_l + 1.0) * (BINS / 2.0)
        s_ref[i * QP:(i + 1) * QP, DHP:DPP] = (raw_r + 1.0) * (BINS / 2.0)
    s_ref[:, DPP:DP] = jnp.full((BB * QP, DP - DPP), -1.0, jnp.float32)

    sv = s_ref[...]
    terms = [lax.dot_general((sv >= float(k)).astype(jnp.bfloat16),
                             w_ref[(k - 1) * DP:k * DP, :],
                             (((1,), (0,)), ((), ())),
                             preferred_element_type=jnp.float32)
             for k in range(1, BINS)]
    while len(terms) > 1:
        terms = [terms[i] + terms[i + 1] for i in range(0, len(terms) - 1, 2)] \
            + ([terms[-1]] if len(terms) % 2 else [])
    lane = lax.broadcasted_iota(jnp.int32, (1, 32), 1)
    cadj = jnp.where(lane == 0, float(DPP),
                     jnp.where(lane == 15, -float(DPP - D), 0.0))
    hist = terms[0] + cadj

    h = jnp.log1p(hist)
    m1 = jnp.tanh(
        lax.dot_general(h, mW1_ref[...], (((1,), (0,)), ((), ())),
                        preferred_element_type=jnp.float32) + mb1_ref[...])
    cls = jnp.tanh(
        lax.dot_general(m1, mW2_ref[...], (((1,), (0,)), ((), ())),
                        preferred_element_type=jnp.float32)[:, 0:1]
        + mb2_ref[...])

    g1 = jnp.tanh(
        qm * lax.dot_general(qe, gW1_ref[...], (((1,), (0,)), ((), ())),
                             preferred_element_type=jnp.float32)
        + gb1_ref[...])
    graw = jnp.tanh(
        lax.dot_general(g1, gW2_ref[...], (((1,), (0,)), ((), ())),
                        preferred_element_type=jnp.float32)[:, 0:1]
        + gb2_ref[...])

    for i in range(BB):
        gr = graw[i * QP:(i + 1) * QP, :]
        qmi = qm[i * QP:(i + 1) * QP, :]
        xm = jnp.where(qmi > 0.0, gr, NEG_BIG)
        xmax = jnp.max(xm, axis=0, keepdims=True)
        ex = jnp.exp(gr - xmax) * qmi
        gate = ex / jnp.sum(ex, axis=0, keepdims=True)
        ci = cls[i * QP:(i + 1) * QP, :]
        out_ref[i:i + 1, :] = jnp.sum(ci * gate, axis=0, keepdims=True)


@jax.jit
def _tc_compute(qe2, dep, qt2, dtp, sel2, rhs2, wsgn, mW1p, mb1p, mW2p,
                mb2p, gW1, gb1p, gW2p, gb2p):
    nsteps = B // BB

    def wspec(r, c):
        return pl.BlockSpec((r, c), lambda i: (0, 0))

    return pl.pallas_call(
        _tc_body,
        grid=(nsteps,),
        in_specs=[
            pl.BlockSpec((BB * QP, E), lambda i: (i, 0)),
            pl.BlockSpec((BB * DHP, 2 * E), lambda i: (i, 0)),
            pl.BlockSpec((BB * QP, 1), lambda i: (i, 0)),
            pl.BlockSpec((BB * DHP, 2), lambda i: (i, 0)),
            wspec(2, 2 * E), wspec(2 * E, 8), wspec((BINS - 1) * DP, 32),
            wspec(32, 32), wspec(1, 32), wspec(32, 8), wspec(1, 1),
            wspec(E, E), wspec(1, E), wspec(E, 8), wspec(1, 1),
        ],
        out_specs=pl.BlockSpec((BB, 1), lambda i: (i, 0)),
        out_shape=jax.ShapeDtypeStruct((B, 1), jnp.float32),
        scratch_shapes=[pltpu.VMEM((BB * QP, DP), jnp.float32)],
    )(qe2, dep, qt2, dtp, sel2, rhs2, wsgn, mW1p, mb1p, mW2p, mb2p, gW1,
      gb1p, gW2p, gb2p)


def kernel(query_tokens, document_tokens, table, mW1, mb1, mW2, mb2,
           gW1, gb1, gW2, gb2):
    qtp = jnp.pad(query_tokens, ((0, 0), (0, QP - Q)))
    dtpad = jnp.pad(document_tokens, ((0, 0), (0, DPP - D)))
    qpad_rows = (jnp.arange(B * (QP - Q), dtype=jnp.int32) % V).reshape(
        B, QP - Q)
    dpad_rows = (jnp.arange(B * (DPP - D), dtype=jnp.int32) % V).reshape(
        B, DPP - D)
    qidx = jnp.concatenate([query_tokens, qpad_rows], axis=1).reshape(-1)
    didx = jnp.concatenate([document_tokens, dpad_rows], axis=1).reshape(-1)
    didx = didx.reshape(-1, GCHUNK // 2, 2).transpose(0, 2, 1).reshape(-1)

    qe2, dep = _sc_gather(qidx, didx, table)
    qt2 = qtp.reshape(B * QP, 1)
    dtp = dtpad.reshape(ND_ROWS // 2, 2)

    lane128 = jnp.arange(2 * E)
    sel2 = jnp.stack([(lane128 < E).astype(jnp.float32),
                      (lane128 >= E).astype(jnp.float32)])
    rhs2 = jnp.zeros((2 * E, 8), jnp.float32)
    rhs2 = rhs2.at[:, 0].set((lane128 < E).astype(jnp.float32))
    rhs2 = rhs2.at[:, 1].set((lane128 >= E).astype(jnp.float32))

    kk = jnp.arange(1, BINS)[:, None, None]
    lane32 = jnp.arange(32)[None, None, :]
    wsgn = jnp.where(lane32 == kk, 1.0,
                     jnp.where(lane32 == kk - 1, -1.0, 0.0))
    wsgn = jnp.broadcast_to(wsgn, (BINS - 1, DP, 32)).reshape(
        (BINS - 1) * DP, 32).astype(jnp.bfloat16)

    mW1p = jnp.zeros((32, 32), jnp.float32).at[:BINS, :BINS].set(mW1)
    mb1p = jnp.zeros((1, 32), jnp.float32).at[0, :BINS].set(mb1)
    mW2p = jnp.zeros((32, 8), jnp.float32).at[:BINS, 0].set(mW2[:, 0])
    mb2p = mb2.reshape(1, 1)
    gb1p = gb1.reshape(1, E)
    gW2p = jnp.zeros((E, 8), jnp.float32).at[:, 0].set(gW2[:, 0])
    gb2p = gb2.reshape(1, 1)

    return _tc_compute(qe2, dep, qt2, dtp, sel2, rhs2, wsgn, mW1p, mb1p,
                       mW2p, mb2p, gW1, gb1p, gW2p, gb2p)


# speedup vs baseline: 1.0302x; 1.0302x over previous
import jax
import jax.numpy as jnp
from jax import lax
from jax.experimental import pallas as pl
from jax.experimental.pallas import tpu as pltpu
from jax.experimental.pallas import tpu_sc as plsc

V = 1000000
E = 64
BINS = 30
B = 4096
Q = 20
QP = 24
D = 200
DP = 256

NQ_ROWS = B * QP
ND_ROWS = B * D
NW = 32
QROWS_PER_W = NQ_ROWS // NW
DROWS_PER_W = ND_ROWS // NW
GCHUNK = 1024
NQ_CHUNKS = QROWS_PER_W // GCHUNK
ND_CHUNKS = DROWS_PER_W // GCHUNK

BB = 8
NEG_BIG = -1e30


def _sc_gather_body(qidx_hbm, didx_hbm, table_hbm, outq_hbm, outd_hbm,
                    idx_v, rows_v, sem):
    wid = lax.axis_index("s") * 2 + lax.axis_index("c")

    def qchunk(i, carry):
        off = wid * QROWS_PER_W + i * GCHUNK
        pltpu.sync_copy(qidx_hbm.at[pl.ds(off, GCHUNK)], idx_v)
        pltpu.async_copy(table_hbm.at[idx_v], rows_v, sem).wait()
        pltpu.sync_copy(rows_v, outq_hbm.at[pl.ds(off, GCHUNK)])
        return carry

    def dchunk(i, carry):
        off = wid * DROWS_PER_W + i * GCHUNK
        pltpu.sync_copy(didx_hbm.at[pl.ds(off, GCHUNK)], idx_v)
        pltpu.async_copy(table_hbm.at[idx_v], rows_v, sem).wait()
        pltpu.sync_copy(rows_v, outd_hbm.at[pl.ds(off, GCHUNK)])
        return carry

    lax.fori_loop(0, NQ_CHUNKS, qchunk, 0, unroll=False)
    lax.fori_loop(0, ND_CHUNKS, dchunk, 0, unroll=False)


@jax.jit
def _sc_gather(qidx, didx, table):
    mesh = plsc.VectorSubcoreMesh(core_axis_name="c", subcore_axis_name="s")
    f = pl.kernel(
        _sc_gather_body,
        out_type=(
            jax.ShapeDtypeStruct((NQ_ROWS, E), jnp.float32),
            jax.ShapeDtypeStruct((ND_ROWS, E), jnp.float32),
        ),
        mesh=mesh,
        compiler_params=pltpu.CompilerParams(use_tc_tiling_on_sc=False),
        scratch_types=[
            pltpu.VMEM((GCHUNK,), jnp.int32),
            pltpu.VMEM((GCHUNK, E), jnp.float32),
            pltpu.SemaphoreType.DMA,
        ],
    )
    return f(qidx, didx, table)


def _tc_body(qe_ref, de_ref, qt_ref, dtr_ref, w_ref, mW1_ref, mb1_ref,
             mW2_ref, mb2_ref, gW1_ref, gb1_ref, gW2_ref, gb2_ref, out_ref,
             s_ref):
    qm = (qt_ref[...] > 1).astype(jnp.float32)
    dmr = (dtr_ref[0] > 1).astype(jnp.float32)
    qe = qe_ref[...]
    de = de_ref[...]

    ones_e = jnp.ones((E, 8), jnp.float32)
    ones_1e = jnp.ones((1, E), jnp.float32)
    qnorm2 = lax.dot_general(qe * qe, ones_e, (((1,), (0,)), ((), ())),
                             preferred_element_type=jnp.float32)[:, 0:1]
    rq = qm * (1.0 / jnp.maximum(jnp.sqrt(qnorm2), 1e-13))
    qn = qe * lax.dot_general(rq, ones_1e, (((1,), (0,)), ((), ())),
                              preferred_element_type=jnp.float32)

    ones_8e = jnp.ones((8, E), jnp.float32)
    dnorm2r = lax.dot_general(ones_8e, de * de, (((1,), (1,)), ((), ())),
                              preferred_element_type=jnp.float32)[0:1, :]
    rdr = dmr * (1.0 / jnp.maximum(jnp.sqrt(dnorm2r), 1e-13))

    for i in range(BB):
        qni = qn[i * QP:(i + 1) * QP, :]
        dei = de[i * D:(i + 1) * D, :]
        raw = lax.dot_general(qni, dei, (((1,), (1,)), ((), ())),
                              preferred_element_type=jnp.float32,
                              precision=lax.Precision.HIGHEST)
        cos = raw * rdr[:, i * D:(i + 1) * D]
        s_ref[i * QP:(i + 1) * QP, 0:D] = (cos + 1.0) * (BINS / 2.0)
    s_ref[:, D:DP] = jnp.full((BB * QP, DP - D), -1.0, jnp.float32)

    sv = s_ref[...]
    terms = [lax.dot_general((sv >= float(k)).astype(jnp.bfloat16),
                             w_ref[(k - 1) * DP:k * DP, :],
                             (((1,), (0,)), ((), ())),
                             preferred_element_type=jnp.float32)
             for k in range(1, BINS)]
    while len(terms) > 1:
        terms = [terms[i] + terms[i + 1] for i in range(0, len(terms) - 1, 2)] \
            + ([terms[-1]] if len(terms) % 2 else [])
    lane = lax.broadcasted_iota(jnp.int32, (1, 32), 1)
    hist = terms[0] + jnp.where(lane == 0, float(D), 0.0)

    h = jnp.log1p(hist)
    m1 = jnp.tanh(
        lax.dot_general(h, mW1_ref[...], (((1,), (0,)), ((), ())),
                        preferred_element_type=jnp.float32) + mb1_ref[...])
    cls = jnp.tanh(
        lax.dot_general(m1, mW2_ref[...], (((1,), (0,)), ((), ())),
                        preferred_element_type=jnp.float32)[:, 0:1]
        + mb2_ref[...])

    g1 = jnp.tanh(
        qm * lax.dot_general(qe, gW1_ref[...], (((1,), (0,)), ((), ())),
                             preferred_element_type=jnp.float32)
        + gb1_ref[...])
    graw = jnp.tanh(
        lax.dot_general(g1, gW2_ref[...], (((1,), (0,)), ((), ())),
                        preferred_element_type=jnp.float32)[:, 0:1]
        + gb2_ref[...])

    for i in range(BB):
        gr = graw[i * QP:(i + 1) * QP, :]
        qmi = qm[i * QP:(i + 1) * QP, :]
        xm = jnp.where(qmi > 0.0, gr, NEG_BIG)
        xmax = jnp.max(xm, axis=0, keepdims=True)
        ex = jnp.exp(gr - xmax) * qmi
        gate = ex / jnp.sum(ex, axis=0, keepdims=True)
        ci = cls[i * QP:(i + 1) * QP, :]
        out_ref[i:i + 1, :] = jnp.sum(ci * gate, axis=0, keepdims=True)


@jax.jit
def _tc_compute(qe2, de2, qt2, dtr, wsgn, mW1p, mb1p, mW2p, mb2p, gW1,
                gb1p, gW2p, gb2p):
    nsteps = B // BB

    def wspec(r, c):
        return pl.BlockSpec((r, c), lambda i: (0, 0))

    return pl.pallas_call(
        _tc_body,
        grid=(nsteps,),
        in_specs=[
            pl.BlockSpec((BB * QP, E), lambda i: (i, 0)),
            pl.BlockSpec((BB * D, E), lambda i: (i, 0)),
            pl.BlockSpec((BB * QP, 1), lambda i: (i, 0)),
            pl.BlockSpec((1, 1, BB * D), lambda i: (i, 0, 0)),
            wspec((BINS - 1) * DP, 32),
            wspec(32, 32), wspec(1, 32), wspec(32, 8), wspec(1, 1),
            wspec(E, E), wspec(1, E), wspec(E, 8), wspec(1, 1),
        ],
        out_specs=pl.BlockSpec((BB, 1), lambda i: (i, 0)),
        out_shape=jax.ShapeDtypeStruct((B, 1), jnp.float32),
        scratch_shapes=[pltpu.VMEM((BB * QP, DP), jnp.float32)],
    )(qe2, de2, qt2, dtr, wsgn, mW1p, mb1p, mW2p, mb2p, gW1, gb1p, gW2p,
      gb2p)


def kernel(query_tokens, document_tokens, table, mW1, mb1, mW2, mb2,
           gW1, gb1, gW2, gb2):
    qtp = jnp.pad(query_tokens, ((0, 0), (0, QP - Q)))
    qpad_rows = (jnp.arange(B * (QP - Q), dtype=jnp.int32) % V).reshape(
        B, QP - Q)
    qidx = jnp.concatenate([query_tokens, qpad_rows], axis=1).reshape(-1)
    didx = document_tokens.reshape(-1)

    qe2, de2 = _sc_gather(qidx, didx, table)
    qt2 = qtp.reshape(B * QP, 1)
    dtr = document_tokens.reshape(B // BB, 1, BB * D)

    kk = jnp.arange(1, BINS)[:, None, None]
    lane32 = jnp.arange(32)[None, None, :]
    wsgn = jnp.where(lane32 == kk, 1.0,
                     jnp.where(lane32 == kk - 1, -1.0, 0.0))
    wsgn = jnp.broadcast_to(wsgn, (BINS - 1, DP, 32)).reshape(
        (BINS - 1) * DP, 32).astype(jnp.bfloat16)

    mW1p = jnp.zeros((32, 32), jnp.float32).at[:BINS, :BINS].set(mW1)
    mb1p = jnp.zeros((1, 32), jnp.float32).at[0, :BINS].set(mb1)
    mW2p = jnp.zeros((32, 8), jnp.float32).at[:BINS, 0].set(mW2[:, 0])
    mb2p = mb2.reshape(1, 1)
    gb1p = gb1.reshape(1, E)
    gW2p = jnp.zeros((E, 8), jnp.float32).at[:, 0].set(gW2[:, 0])
    gb2p = gb2.reshape(1, 1)

    return _tc_compute(qe2, de2, qt2, dtr, wsgn, mW1p, mb1p, mW2p, mb2p,
                       gW1, gb1p, gW2p, gb2p)


# speedup vs baseline: 1.2896x; 1.2518x over previous
import jax
import jax.numpy as jnp
from jax import lax
from jax.experimental import pallas as pl
from jax.experimental.pallas import tpu as pltpu
from jax.experimental.pallas import tpu_sc as plsc

V = 1000000
E = 64
BINS = 30
B = 4096
Q = 20
QP = 24
D = 200
DP = 256

NQ_ROWS = B * QP
ND_ROWS = B * D
NW = 32
QROWS_PER_W = NQ_ROWS // NW
DROWS_PER_W = ND_ROWS // NW
GCHUNK = 1024
NQ_CHUNKS = QROWS_PER_W // GCHUNK
ND_CHUNKS = DROWS_PER_W // GCHUNK

BB = 8
NEG_BIG = -1e30


def _sc_gather_body(qidx_hbm, didx_hbm, table_hbm, outq_hbm, outd_hbm,
                    idx_v, rows_v, sem):
    wid = lax.axis_index("s") * 2 + lax.axis_index("c")

    def qchunk(i, carry):
        off = wid * QROWS_PER_W + i * GCHUNK
        pltpu.sync_copy(qidx_hbm.at[pl.ds(off, GCHUNK)], idx_v)
        pltpu.async_copy(table_hbm.at[idx_v], rows_v, sem).wait()
        pltpu.sync_copy(rows_v, outq_hbm.at[pl.ds(off, GCHUNK)])
        return carry

    def dchunk(i, carry):
        off = wid * DROWS_PER_W + i * GCHUNK
        pltpu.sync_copy(didx_hbm.at[pl.ds(off, GCHUNK)], idx_v)
        pltpu.async_copy(table_hbm.at[idx_v], rows_v, sem).wait()
        pltpu.sync_copy(rows_v, outd_hbm.at[pl.ds(off, GCHUNK)])
        return carry

    lax.fori_loop(0, NQ_CHUNKS, qchunk, 0, unroll=False)
    lax.fori_loop(0, ND_CHUNKS, dchunk, 0, unroll=False)


@jax.jit
def _sc_gather(qidx, didx, table):
    mesh = plsc.VectorSubcoreMesh(core_axis_name="c", subcore_axis_name="s")
    f = pl.kernel(
        _sc_gather_body,
        out_type=(
            jax.ShapeDtypeStruct((NQ_ROWS, E), jnp.float32),
            jax.ShapeDtypeStruct((ND_ROWS, E), jnp.float32),
        ),
        mesh=mesh,
        compiler_params=pltpu.CompilerParams(use_tc_tiling_on_sc=False),
        scratch_types=[
            pltpu.VMEM((GCHUNK,), jnp.int32),
            pltpu.VMEM((GCHUNK, E), jnp.float32),
            pltpu.SemaphoreType.DMA,
        ],
    )
    return f(qidx, didx, table)


def _tc_body(qe_ref, de_ref, qt_ref, dtr_ref, w_ref, mW1_ref, mb1_ref,
             mW2_ref, mb2_ref, gW1_ref, gb1_ref, gW2_ref, gb2_ref, out_ref,
             s_ref):
    qm = (qt_ref[...] > 1).astype(jnp.float32)
    dmr = (dtr_ref[0] > 1).astype(jnp.float32)
    qe = qe_ref[...]
    de = de_ref[...]

    ones_e = jnp.ones((E, 8), jnp.float32)
    ones_1e = jnp.ones((1, E), jnp.float32)
    qnorm2 = lax.dot_general(qe * qe, ones_e, (((1,), (0,)), ((), ())),
                             preferred_element_type=jnp.float32)[:, 0:1]
    rq = qm * (1.0 / jnp.maximum(jnp.sqrt(qnorm2), 1e-13))
    qn = qe * lax.dot_general(rq, ones_1e, (((1,), (0,)), ((), ())),
                              preferred_element_type=jnp.float32)

    ones_8e = jnp.ones((8, E), jnp.float32)
    dnorm2r = lax.dot_general(ones_8e, de * de, (((1,), (1,)), ((), ())),
                              preferred_element_type=jnp.float32)[0:1, :]
    rdr = dmr * (1.0 / jnp.maximum(jnp.sqrt(dnorm2r), 1e-13))

    for i in range(BB):
        qni = qn[i * QP:(i + 1) * QP, :]
        dei = de[i * D:(i + 1) * D, :]
        raw = lax.dot_general(qni, dei, (((1,), (1,)), ((), ())),
                              preferred_element_type=jnp.float32)
        cos = raw * rdr[:, i * D:(i + 1) * D]
        s_ref[i * QP:(i + 1) * QP, 0:D] = (cos + 1.0) * (BINS / 2.0)
    s_ref[:, D:DP] = jnp.full((BB * QP, DP - D), -1.0, jnp.float32)

    sv = s_ref[...]
    terms = [lax.dot_general((sv >= float(k)).astype(jnp.bfloat16),
                             w_ref[(k - 1) * DP:k * DP, :],
                             (((1,), (0,)), ((), ())),
                             preferred_element_type=jnp.float32)
             for k in range(1, BINS)]
    while len(terms) > 1:
        terms = [terms[i] + terms[i + 1] for i in range(0, len(terms) - 1, 2)] \
            + ([terms[-1]] if len(terms) % 2 else [])
    lane = lax.broadcasted_iota(jnp.int32, (1, 32), 1)
    hist = terms[0] + jnp.where(lane == 0, float(D), 0.0)

    h = jnp.log1p(hist)
    m1 = jnp.tanh(
        lax.dot_general(h, mW1_ref[...], (((1,), (0,)), ((), ())),
                        preferred_element_type=jnp.float32) + mb1_ref[...])
    cls = jnp.tanh(
        lax.dot_general(m1, mW2_ref[...], (((1,), (0,)), ((), ())),
                        preferred_element_type=jnp.float32)[:, 0:1]
        + mb2_ref[...])

    g1 = jnp.tanh(
        qm * lax.dot_general(qe, gW1_ref[...], (((1,), (0,)), ((), ())),
                             preferred_element_type=jnp.float32)
        + gb1_ref[...])
    graw = jnp.tanh(
        lax.dot_general(g1, gW2_ref[...], (((1,), (0,)), ((), ())),
                        preferred_element_type=jnp.float32)[:, 0:1]
        + gb2_ref[...])

    for i in range(BB):
        gr = graw[i * QP:(i + 1) * QP, :]
        qmi = qm[i * QP:(i + 1) * QP, :]
        xm = jnp.where(qmi > 0.0, gr, NEG_BIG)
        xmax = jnp.max(xm, axis=0, keepdims=True)
        ex = jnp.exp(gr - xmax) * qmi
        gate = ex / jnp.sum(ex, axis=0, keepdims=True)
        ci = cls[i * QP:(i + 1) * QP, :]
        out_ref[i:i + 1, :] = jnp.sum(ci * gate, axis=0, keepdims=True)


@jax.jit
def _tc_compute(qe2, de2, qt2, dtr, wsgn, mW1p, mb1p, mW2p, mb2p, gW1,
                gb1p, gW2p, gb2p):
    nsteps = B // BB

    def wspec(r, c):
        return pl.BlockSpec((r, c), lambda i: (0, 0))

    return pl.pallas_call(
        _tc_body,
        grid=(nsteps,),
        in_specs=[
            pl.BlockSpec((BB * QP, E), lambda i: (i, 0)),
            pl.BlockSpec((BB * D, E), lambda i: (i, 0)),
            pl.BlockSpec((BB * QP, 1), lambda i: (i, 0)),
            pl.BlockSpec((1, 1, BB * D), lambda i: (i, 0, 0)),
            wspec((BINS - 1) * DP, 32),
            wspec(32, 32), wspec(1, 32), wspec(32, 8), wspec(1, 1),
            wspec(E, E), wspec(1, E), wspec(E, 8), wspec(1, 1),
        ],
        out_specs=pl.BlockSpec((BB, 1), lambda i: (i, 0)),
        out_shape=jax.ShapeDtypeStruct((B, 1), jnp.float32),
        scratch_shapes=[pltpu.VMEM((BB * QP, DP), jnp.float32)],
    )(qe2, de2, qt2, dtr, wsgn, mW1p, mb1p, mW2p, mb2p, gW1, gb1p, gW2p,
      gb2p)


def kernel(query_tokens, document_tokens, table, mW1, mb1, mW2, mb2,
           gW1, gb1, gW2, gb2):
    qtp = jnp.pad(query_tokens, ((0, 0), (0, QP - Q)))
    qpad_rows = (jnp.arange(B * (QP - Q), dtype=jnp.int32) % V).reshape(
        B, QP - Q)
    qidx = jnp.concatenate([query_tokens, qpad_rows], axis=1).reshape(-1)
    didx = document_tokens.reshape(-1)

    qe2, de2 = _sc_gather(qidx, didx, table)
    qt2 = qtp.reshape(B * QP, 1)
    dtr = document_tokens.reshape(B // BB, 1, BB * D)

    kk = jnp.arange(1, BINS)[:, None, None]
    lane32 = jnp.arange(32)[None, None, :]
    wsgn = jnp.where(lane32 == kk, 1.0,
                     jnp.where(lane32 == kk - 1, -1.0, 0.0))
    wsgn = jnp.broadcast_to(wsgn, (BINS - 1, DP, 32)).reshape(
        (BINS - 1) * DP, 32).astype(jnp.bfloat16)

    mW1p = jnp.zeros((32, 32), jnp.float32).at[:BINS, :BINS].set(mW1)
    mb1p = jnp.zeros((1, 32), jnp.float32).at[0, :BINS].set(mb1)
    mW2p = jnp.zeros((32, 8), jnp.float32).at[:BINS, 0].set(mW2[:, 0])
    mb2p = mb2.reshape(1, 1)
    gb1p = gb1.reshape(1, E)
    gW2p = jnp.zeros((E, 8), jnp.float32).at[:, 0].set(gW2[:, 0])
    gb2p = gb2.reshape(1, 1)

    return _tc_compute(qe2, de2, qt2, dtr, wsgn, mW1p, mb1p, mW2p, mb2p,
                       gW1, gb1p, gW2p, gb2p)


# speedup vs baseline: 1.2900x; 1.0004x over previous
import jax
import jax.numpy as jnp
from jax import lax
from jax.experimental import pallas as pl
from jax.experimental.pallas import tpu as pltpu
from jax.experimental.pallas import tpu_sc as plsc

V = 1000000
E = 64
BINS = 30
B = 4096
Q = 20
QP = 24
D = 200
DP = 256

NQ_ROWS = B * QP
ND_ROWS = B * D
NW = 32
QROWS_PER_W = NQ_ROWS // NW
DROWS_PER_W = ND_ROWS // NW
GCHUNK = 1024
NQ_CHUNKS = QROWS_PER_W // GCHUNK
ND_CHUNKS = DROWS_PER_W // GCHUNK

BB = 8
NEG_BIG = -1e30


def _sc_gather_body(qidx_hbm, didx_hbm, table_hbm, outq_hbm, outd_hbm,
                    idx_v, rows_v, sem):
    wid = lax.axis_index("s") * 2 + lax.axis_index("c")

    def qchunk(i, carry):
        off = wid * QROWS_PER_W + i * GCHUNK
        pltpu.sync_copy(qidx_hbm.at[pl.ds(off, GCHUNK)], idx_v)
        pltpu.async_copy(table_hbm.at[idx_v], rows_v, sem).wait()
        pltpu.sync_copy(rows_v, outq_hbm.at[pl.ds(off, GCHUNK)])
        return carry

    def dchunk(i, carry):
        off = wid * DROWS_PER_W + i * GCHUNK
        pltpu.sync_copy(didx_hbm.at[pl.ds(off, GCHUNK)], idx_v)
        pltpu.async_copy(table_hbm.at[idx_v], rows_v, sem).wait()
        pltpu.sync_copy(rows_v, outd_hbm.at[pl.ds(off, GCHUNK)])
        return carry

    lax.fori_loop(0, NQ_CHUNKS, qchunk, 0, unroll=False)
    lax.fori_loop(0, ND_CHUNKS, dchunk, 0, unroll=False)


@jax.jit
def _sc_gather(qidx, didx, table):
    mesh = plsc.VectorSubcoreMesh(core_axis_name="c", subcore_axis_name="s")
    f = pl.kernel(
        _sc_gather_body,
        out_type=(
            jax.ShapeDtypeStruct((NQ_ROWS, E), jnp.float32),
            jax.ShapeDtypeStruct((ND_ROWS, E), jnp.float32),
        ),
        mesh=mesh,
        compiler_params=pltpu.CompilerParams(use_tc_tiling_on_sc=False),
        scratch_types=[
            pltpu.VMEM((GCHUNK,), jnp.int32),
            pltpu.VMEM((GCHUNK, E), jnp.float32),
            pltpu.SemaphoreType.DMA,
        ],
    )
    return f(qidx, didx, table)


def _tc_body(qe_ref, de_ref, qt_ref, dtr_ref, w_ref, mW1_ref, mb1_ref,
             mW2_ref, mb2_ref, gW1_ref, gb1_ref, gW2_ref, gb2_ref, out_ref,
             s_ref):
    qm = (qt_ref[...] > 1).astype(jnp.float32)
    dmr = (dtr_ref[0] > 1).astype(jnp.float32)
    qe = qe_ref[...]
    de = de_ref[...]

    ones_e = jnp.ones((E, 8), jnp.float32)
    ones_1e = jnp.ones((1, E), jnp.float32)
    qnorm2 = lax.dot_general(qe * qe, ones_e, (((1,), (0,)), ((), ())),
                             preferred_element_type=jnp.float32)[:, 0:1]
    rq = qm * (1.0 / jnp.maximum(jnp.sqrt(qnorm2), 1e-13))
    qn = qe * lax.dot_general(rq, ones_1e, (((1,), (0,)), ((), ())),
                              preferred_element_type=jnp.float32)

    ones_8e = jnp.ones((8, E), jnp.float32)
    dnorm2r = lax.dot_general(ones_8e, de * de, (((1,), (1,)), ((), ())),
                              preferred_element_type=jnp.float32)[0:1, :]
    rdr = dmr * (1.0 / jnp.maximum(jnp.sqrt(dnorm2r), 1e-13))

    for i in range(BB):
        qni = qn[i * QP:(i + 1) * QP, :]
        dei = de[i * D:(i + 1) * D, :]
        raw = lax.dot_general(qni, dei, (((1,), (1,)), ((), ())),
                              preferred_element_type=jnp.float32)
        cos = raw * rdr[:, i * D:(i + 1) * D]
        s_ref[i * QP:(i + 1) * QP, 0:D] = (cos + 1.0) * (BINS / 2.0)
    s_ref[:, D:DP] = jnp.full((BB * QP, DP - D), -1.0, jnp.float32)

    sv = s_ref[...]
    terms = [lax.dot_general((sv >= float(k)).astype(jnp.bfloat16),
                             w_ref[(k - 1) * DP:k * DP, :],
                             (((1,), (0,)), ((), ())),
                             preferred_element_type=jnp.float32)
             for k in range(1, BINS)]
    while len(terms) > 1:
        terms = [terms[i] + terms[i + 1] for i in range(0, len(terms) - 1, 2)] \
            + ([terms[-1]] if len(terms) % 2 else [])
    lane = lax.broadcasted_iota(jnp.int32, (1, 32), 1)
    hist = terms[0] + jnp.where(lane == 0, float(D), 0.0)

    h = jnp.log1p(hist)
    m1 = jnp.tanh(
        lax.dot_general(h, mW1_ref[...], (((1,), (0,)), ((), ())),
                        preferred_element_type=jnp.float32) + mb1_ref[...])
    cls = jnp.tanh(
        lax.dot_general(m1, mW2_ref[...], (((1,), (0,)), ((), ())),
                        preferred_element_type=jnp.float32)[:, 0:1]
        + mb2_ref[...])

    g1 = jnp.tanh(
        qm * lax.dot_general(qe, gW1_ref[...], (((1,), (0,)), ((), ())),
                             preferred_element_type=jnp.float32)
        + gb1_ref[...])
    graw = jnp.tanh(
        lax.dot_general(g1, gW2_ref[...], (((1,), (0,)), ((), ())),
                        preferred_element_type=jnp.float32)[:, 0:1]
        + gb2_ref[...])

    for i in range(BB):
        gr = graw[i * QP:(i + 1) * QP, :]
        qmi = qm[i * QP:(i + 1) * QP, :]
        xm = jnp.where(qmi > 0.0, gr, NEG_BIG)
        xmax = jnp.max(xm, axis=0, keepdims=True)
        ex = jnp.exp(gr - xmax) * qmi
        gate = ex / jnp.sum(ex, axis=0, keepdims=True)
        ci = cls[i * QP:(i + 1) * QP, :]
        out_ref[i:i + 1, :] = jnp.sum(ci * gate, axis=0, keepdims=True)


@jax.jit
def _tc_compute(qe2, de2, qt2, dtr, wsgn, mW1p, mb1p, mW2p, mb2p, gW1,
                gb1p, gW2p, gb2p):
    nsteps = B // BB

    def wspec(r, c):
        return pl.BlockSpec((r, c), lambda i: (0, 0))

    return pl.pallas_call(
        _tc_body,
        grid=(nsteps,),
        in_specs=[
            pl.BlockSpec((BB * QP, E), lambda i: (i, 0)),
            pl.BlockSpec((BB * D, E), lambda i: (i, 0)),
            pl.BlockSpec((BB * QP, 1), lambda i: (i, 0)),
            pl.BlockSpec((1, 1, BB * D), lambda i: (i, 0, 0)),
            wspec((BINS - 1) * DP, 32),
            wspec(32, 32), wspec(1, 32), wspec(32, 8), wspec(1, 1),
            wspec(E, E), wspec(1, E), wspec(E, 8), wspec(1, 1),
        ],
        out_specs=pl.BlockSpec((BB, 1), lambda i: (i, 0)),
        out_shape=jax.ShapeDtypeStruct((B, 1), jnp.float32),
        scratch_shapes=[pltpu.VMEM((BB * QP, DP), jnp.float32)],
    )(qe2, de2, qt2, dtr, wsgn, mW1p, mb1p, mW2p, mb2p, gW1, gb1p, gW2p,
      gb2p)


def kernel(query_tokens, document_tokens, table, mW1, mb1, mW2, mb2,
           gW1, gb1, gW2, gb2):
    qtp = jnp.pad(query_tokens, ((0, 0), (0, QP - Q)))
    qpad_rows = (jnp.arange(B * (QP - Q), dtype=jnp.int32) % V).reshape(
        B, QP - Q)
    qidx = jnp.concatenate([query_tokens, qpad_rows], axis=1).reshape(-1)
    didx = document_tokens.reshape(-1)

    table_b = lax.optimization_barrier(table)
    qe2, de2 = _sc_gather(qidx, didx, table_b)
    qt2 = qtp.reshape(B * QP, 1)
    dtr = document_tokens.reshape(B // BB, 1, BB * D)

    kk = jnp.arange(1, BINS)[:, None, None]
    lane32 = jnp.arange(32)[None, None, :]
    wsgn = jnp.where(lane32 == kk, 1.0,
                     jnp.where(lane32 == kk - 1, -1.0, 0.0))
    wsgn = jnp.broadcast_to(wsgn, (BINS - 1, DP, 32)).reshape(
        (BINS - 1) * DP, 32).astype(jnp.bfloat16)

    mW1p = jnp.zeros((32, 32), jnp.float32).at[:BINS, :BINS].set(mW1)
    mb1p = jnp.zeros((1, 32), jnp.float32).at[0, :BINS].set(mb1)
    mW2p = jnp.zeros((32, 8), jnp.float32).at[:BINS, 0].set(mW2[:, 0])
    mb2p = mb2.reshape(1, 1)
    gb1p = gb1.reshape(1, E)
    gW2p = jnp.zeros((E, 8), jnp.float32).at[:, 0].set(gW2[:, 0])
    gb2p = gb2.reshape(1, 1)

    return _tc_compute(qe2, de2, qt2, dtr, wsgn, mW1p, mb1p, mW2p, mb2p,
                       gW1, gb1p, gW2p, gb2p)


# speedup vs baseline: 1.4655x; 1.1360x over previous
import jax
import jax.numpy as jnp
from jax import lax
from jax.experimental import pallas as pl
from jax.experimental.pallas import tpu as pltpu
from jax.experimental.pallas import tpu_sc as plsc

V = 1000000
E = 64
BINS = 30
B = 4096
Q = 20
QP = 24
D = 200
DP = 256

NQ_ROWS = B * QP
ND_ROWS = B * D
NW = 32
QROWS_PER_W = NQ_ROWS // NW
DROWS_PER_W = ND_ROWS // NW
GCHUNK = 1024
NQ_CHUNKS = QROWS_PER_W // GCHUNK
ND_CHUNKS = DROWS_PER_W // GCHUNK

BB = 32
NEG_BIG = -1e30


def _sc_gather_body(qidx_hbm, didx_hbm, table_hbm, outq_hbm, outd_hbm,
                    idx_v, rows_v, sem):
    wid = lax.axis_index("s") * 2 + lax.axis_index("c")

    def qchunk(i, carry):
        off = wid * QROWS_PER_W + i * GCHUNK
        pltpu.sync_copy(qidx_hbm.at[pl.ds(off, GCHUNK)], idx_v)
        pltpu.async_copy(table_hbm.at[idx_v], rows_v, sem).wait()
        pltpu.sync_copy(rows_v, outq_hbm.at[pl.ds(off, GCHUNK)])
        return carry

    def dchunk(i, carry):
        off = wid * DROWS_PER_W + i * GCHUNK
        pltpu.sync_copy(didx_hbm.at[pl.ds(off, GCHUNK)], idx_v)
        pltpu.async_copy(table_hbm.at[idx_v], rows_v, sem).wait()
        pltpu.sync_copy(rows_v, outd_hbm.at[pl.ds(off, GCHUNK)])
        return carry

    lax.fori_loop(0, NQ_CHUNKS, qchunk, 0, unroll=False)
    lax.fori_loop(0, ND_CHUNKS, dchunk, 0, unroll=False)


@jax.jit
def _sc_gather(qidx, didx, table):
    mesh = plsc.VectorSubcoreMesh(core_axis_name="c", subcore_axis_name="s")
    f = pl.kernel(
        _sc_gather_body,
        out_type=(
            jax.ShapeDtypeStruct((NQ_ROWS, E), jnp.float32),
            jax.ShapeDtypeStruct((ND_ROWS, E), jnp.float32),
        ),
        mesh=mesh,
        compiler_params=pltpu.CompilerParams(use_tc_tiling_on_sc=False),
        scratch_types=[
            pltpu.VMEM((GCHUNK,), jnp.int32),
            pltpu.VMEM((GCHUNK, E), jnp.float32),
            pltpu.SemaphoreType.DMA,
        ],
    )
    return f(qidx, didx, table)


def _tc_body(qe_ref, de_ref, qt_ref, dtr_ref, w_ref, mW1_ref, mb1_ref,
             mW2_ref, mb2_ref, gW1_ref, gb1_ref, gW2_ref, gb2_ref, out_ref,
             s_ref):
    qm = (qt_ref[...] > 1).astype(jnp.float32)
    dmr = (dtr_ref[0] > 1).astype(jnp.float32)
    qe = qe_ref[...]
    de = de_ref[...]

    ones_e = jnp.ones((E, 8), jnp.float32)
    ones_1e = jnp.ones((1, E), jnp.float32)
    qnorm2 = lax.dot_general(qe * qe, ones_e, (((1,), (0,)), ((), ())),
                             preferred_element_type=jnp.float32)[:, 0:1]
    rq = qm * (1.0 / jnp.maximum(jnp.sqrt(qnorm2), 1e-13))
    qn = qe * lax.dot_general(rq, ones_1e, (((1,), (0,)), ((), ())),
                              preferred_element_type=jnp.float32)

    ones_8e = jnp.ones((8, E), jnp.float32)
    dnorm2r = lax.dot_general(ones_8e, de * de, (((1,), (1,)), ((), ())),
                              preferred_element_type=jnp.float32)[0:1, :]
    rdr = dmr * (1.0 / jnp.maximum(jnp.sqrt(dnorm2r), 1e-13))

    for i in range(BB):
        qni = qn[i * QP:(i + 1) * QP, :]
        dei = de[i * D:(i + 1) * D, :]
        raw = lax.dot_general(qni, dei, (((1,), (1,)), ((), ())),
                              preferred_element_type=jnp.float32)
        cos = raw * rdr[:, i * D:(i + 1) * D]
        s_ref[i * QP:(i + 1) * QP, 0:D] = (cos + 1.0) * (BINS / 2.0)
    s_ref[:, D:DP] = jnp.full((BB * QP, DP - D), -1.0, jnp.float32)

    sv = s_ref[...]
    terms = [lax.dot_general((sv >= float(k)).astype(jnp.bfloat16),
                             w_ref[(k - 1) * DP:k * DP, :],
                             (((1,), (0,)), ((), ())),
                             preferred_element_type=jnp.float32)
             for k in range(1, BINS)]
    while len(terms) > 1:
        terms = [terms[i] + terms[i + 1] for i in range(0, len(terms) - 1, 2)] \
            + ([terms[-1]] if len(terms) % 2 else [])
    lane = lax.broadcasted_iota(jnp.int32, (1, 32), 1)
    hist = terms[0] + jnp.where(lane == 0, float(D), 0.0)

    h = jnp.log1p(hist)
    m1 = jnp.tanh(
        lax.dot_general(h, mW1_ref[...], (((1,), (0,)), ((), ())),
                        preferred_element_type=jnp.float32) + mb1_ref[...])
    cls = jnp.tanh(
        lax.dot_general(m1, mW2_ref[...], (((1,), (0,)), ((), ())),
                        preferred_element_type=jnp.float32)[:, 0:1]
        + mb2_ref[...])

    g1 = jnp.tanh(
        qm * lax.dot_general(qe, gW1_ref[...], (((1,), (0,)), ((), ())),
                             preferred_element_type=jnp.float32)
        + gb1_ref[...])
    graw = jnp.tanh(
        lax.dot_general(g1, gW2_ref[...], (((1,), (0,)), ((), ())),
                        preferred_element_type=jnp.float32)[:, 0:1]
        + gb2_ref[...])

    for i in range(BB):
        gr = graw[i * QP:(i + 1) * QP, :]
        qmi = qm[i * QP:(i + 1) * QP, :]
        xm = jnp.where(qmi > 0.0, gr, NEG_BIG)
        xmax = jnp.max(xm, axis=0, keepdims=True)
        ex = jnp.exp(gr - xmax) * qmi
        gate = ex / jnp.sum(ex, axis=0, keepdims=True)
        ci = cls[i * QP:(i + 1) * QP, :]
        out_ref[i:i + 1, :] = jnp.sum(ci * gate, axis=0, keepdims=True)


@jax.jit
def _tc_compute(qe2, de2, qt2, dtr, wsgn, mW1p, mb1p, mW2p, mb2p, gW1,
                gb1p, gW2p, gb2p):
    nsteps = B // BB

    def wspec(r, c):
        return pl.BlockSpec((r, c), lambda i: (0, 0))

    return pl.pallas_call(
        _tc_body,
        grid=(nsteps,),
        in_specs=[
            pl.BlockSpec((BB * QP, E), lambda i: (i, 0)),
            pl.BlockSpec((BB * D, E), lambda i: (i, 0)),
            pl.BlockSpec((BB * QP, 1), lambda i: (i, 0)),
            pl.BlockSpec((1, 1, BB * D), lambda i: (i, 0, 0)),
            wspec((BINS - 1) * DP, 32),
            wspec(32, 32), wspec(1, 32), wspec(32, 8), wspec(1, 1),
            wspec(E, E), wspec(1, E), wspec(E, 8), wspec(1, 1),
        ],
        out_specs=pl.BlockSpec((BB, 1), lambda i: (i, 0)),
        out_shape=jax.ShapeDtypeStruct((B, 1), jnp.float32),
        scratch_shapes=[pltpu.VMEM((BB * QP, DP), jnp.float32)],
    )(qe2, de2, qt2, dtr, wsgn, mW1p, mb1p, mW2p, mb2p, gW1, gb1p, gW2p,
      gb2p)


def kernel(query_tokens, document_tokens, table, mW1, mb1, mW2, mb2,
           gW1, gb1, gW2, gb2):
    qtp = jnp.pad(query_tokens, ((0, 0), (0, QP - Q)))
    qpad_rows = (jnp.arange(B * (QP - Q), dtype=jnp.int32) % V).reshape(
        B, QP - Q)
    qidx = jnp.concatenate([query_tokens, qpad_rows], axis=1).reshape(-1)
    didx = document_tokens.reshape(-1)

    qe2, de2 = _sc_gather(qidx, didx, table)
    qt2 = qtp.reshape(B * QP, 1)
    dtr = document_tokens.reshape(B // BB, 1, BB * D)

    kk = jnp.arange(1, BINS)[:, None, None]
    lane32 = jnp.arange(32)[None, None, :]
    wsgn = jnp.where(lane32 == kk, 1.0,
                     jnp.where(lane32 == kk - 1, -1.0, 0.0))
    wsgn = jnp.broadcast_to(wsgn, (BINS - 1, DP, 32)).reshape(
        (BINS - 1) * DP, 32).astype(jnp.bfloat16)

    mW1p = jnp.zeros((32, 32), jnp.float32).at[:BINS, :BINS].set(mW1)
    mb1p = jnp.zeros((1, 32), jnp.float32).at[0, :BINS].set(mb1)
    mW2p = jnp.zeros((32, 8), jnp.float32).at[:BINS, 0].set(mW2[:, 0])
    mb2p = mb2.reshape(1, 1)
    gb1p = gb1.reshape(1, E)
    gW2p = jnp.zeros((E, 8), jnp.float32).at[:, 0].set(gW2[:, 0])
    gb2p = gb2.reshape(1, 1)

    return _tc_compute(qe2, de2, qt2, dtr, wsgn, mW1p, mb1p, mW2p, mb2p,
                       gW1, gb1p, gW2p, gb2p)
